# baseline (speedup 1.0000x reference)
import functools

import jax
import jax.numpy as jnp
from jax import lax
from jax.experimental import pallas as pl
from jax.experimental.pallas import tpu as pltpu
from jax.experimental.pallas import tpu_sc as plsc

_N_CLASS = 100000
_DIM = 16
_N_STEP = 20
_BATCH = 1024

_TOTAL = _BATCH * _N_STEP
_NW = 32
_B_PER_W = _TOTAL // _NW
_CHUNK = 128
_N_CHUNKS = _B_PER_W // _CHUNK


def _make_sc_gather():
    mesh = plsc.VectorSubcoreMesh(core_axis_name="c", subcore_axis_name="s")

    @functools.partial(
        pl.kernel,
        mesh=mesh,
        out_type=jax.ShapeDtypeStruct((_TOTAL, _DIM), jnp.float32),
        scratch_types=[
            pltpu.VMEM((_B_PER_W,), jnp.int32),
            pltpu.VMEM((_B_PER_W, _DIM), jnp.float32),
            pltpu.SemaphoreType.DMA,
        ],
        compiler_params=pltpu.CompilerParams(use_tc_tiling_on_sc=False),
    )
    def gather_kernel(table_hbm, idx_hbm, out_hbm, idx_v, rows_v, sem):
        wid = lax.axis_index("s") * 2 + lax.axis_index("c")
        base = wid * _B_PER_W
        pltpu.sync_copy(idx_hbm.at[pl.ds(base, _B_PER_W)], idx_v)
        copies = []
        for j in range(_N_CHUNKS):
            copies.append(pltpu.async_copy(
                table_hbm.at[idx_v.at[pl.ds(j * _CHUNK, _CHUNK)]],
                rows_v.at[pl.ds(j * _CHUNK, _CHUNK)],
                sem,
            ))
        for c in copies:
            c.wait()
        pltpu.sync_copy(rows_v, out_hbm.at[pl.ds(base, _B_PER_W)])

    return gather_kernel


_sc_gather = _make_sc_gather()


_BN = 1024


def _proj_body(flat_ref, w_ref, b_ref, out_ref):
    out_ref[...] = lax.dot_general(
        flat_ref[...], w_ref[...],
        (((1,), (1,)), ((), ())),
        preferred_element_type=jnp.float32,
    ) + b_ref[...]


@jax.jit
def _projection(flat, W, b2):
    nb = pl.cdiv(_N_CLASS, _BN)
    return pl.pallas_call(
        _proj_body,
        grid=(nb,),
        in_specs=[
            pl.BlockSpec((_BATCH, _N_STEP * _DIM), lambda i: (0, 0)),
            pl.BlockSpec((_BN, _N_STEP * _DIM), lambda i: (i, 0)),
            pl.BlockSpec((1, _BN), lambda i: (0, i)),
        ],
        out_specs=pl.BlockSpec((_BATCH, _BN), lambda i: (0, i)),
        out_shape=jax.ShapeDtypeStruct((_BATCH, _N_CLASS), jnp.float32),
        compiler_params=pltpu.CompilerParams(
            dimension_semantics=("parallel",),
        ),
    )(flat, W, b2)


def kernel(x, table, W, b):
    idx = x.reshape(-1).astype(jnp.int32)
    rows = _sc_gather(table, idx)
    flat = rows.reshape(_BATCH, _N_STEP * _DIM)
    return _projection(flat, W, b.reshape(1, _N_CLASS))


# speedup vs baseline: 2.3201x; 2.3201x over previous
import functools

import jax
import jax.numpy as jnp
from jax import lax
from jax.experimental import pallas as pl
from jax.experimental.pallas import tpu as pltpu
from jax.experimental.pallas import tpu_sc as plsc

_N_CLASS = 100000
_DIM = 16
_N_STEP = 20
_BATCH = 1024

_TOTAL = _BATCH * _N_STEP
_NW = 32
_B_PER_W = _TOTAL // _NW
_CHUNK = 128
_N_CHUNKS = _B_PER_W // _CHUNK


def _make_sc_gather():
    mesh = plsc.VectorSubcoreMesh(core_axis_name="c", subcore_axis_name="s")

    @functools.partial(
        pl.kernel,
        mesh=mesh,
        out_type=jax.ShapeDtypeStruct((_TOTAL, _DIM), jnp.float32),
        scratch_types=[
            pltpu.VMEM((_B_PER_W,), jnp.int32),
            pltpu.VMEM((_B_PER_W, _DIM), jnp.float32),
            pltpu.SemaphoreType.DMA,
        ],
        compiler_params=pltpu.CompilerParams(use_tc_tiling_on_sc=False),
    )
    def gather_kernel(table_hbm, idx_hbm, out_hbm, idx_v, rows_v, sem):
        wid = lax.axis_index("s") * 2 + lax.axis_index("c")
        base = wid * _B_PER_W
        pltpu.sync_copy(idx_hbm.at[pl.ds(base, _B_PER_W)], idx_v)
        copies = []
        for j in range(_N_CHUNKS):
            copies.append(pltpu.async_copy(
                table_hbm.at[idx_v.at[pl.ds(j * _CHUNK, _CHUNK)]],
                rows_v.at[pl.ds(j * _CHUNK, _CHUNK)],
                sem,
            ))
        for c in copies:
            c.wait()
        pltpu.sync_copy(rows_v, out_hbm.at[pl.ds(base, _B_PER_W)])

    return gather_kernel


_sc_gather = _make_sc_gather()


_BN = 1024


def _proj_body(wt_ref, flat_ref, b_ref, out_ref):
    out_ref[...] = lax.dot_general(
        wt_ref[...], flat_ref[...],
        (((0,), (1,)), ((), ())),
        preferred_element_type=jnp.float32,
    ) + b_ref[...]


@jax.jit
def _projection(Wt, flat, b2):
    nb = pl.cdiv(_N_CLASS, _BN)
    return pl.pallas_call(
        _proj_body,
        grid=(nb,),
        in_specs=[
            pl.BlockSpec((_N_STEP * _DIM, _BN), lambda i: (0, i)),
            pl.BlockSpec((_BATCH, _N_STEP * _DIM), lambda i: (0, 0)),
            pl.BlockSpec((_BN, 1), lambda i: (i, 0)),
        ],
        out_specs=pl.BlockSpec((_BN, _BATCH), lambda i: (i, 0)),
        out_shape=jax.ShapeDtypeStruct((_N_CLASS, _BATCH), jnp.float32),
        compiler_params=pltpu.CompilerParams(
            dimension_semantics=("parallel",),
        ),
    )(Wt, flat, b2)


def kernel(x, table, W, b):
    idx = x.reshape(-1).astype(jnp.int32)
    rows = _sc_gather(table, idx)
    flat = rows.reshape(_BATCH, _N_STEP * _DIM)
    outT = _projection(W.T, flat, b.reshape(_N_CLASS, 1))
    return outT.T


# speedup vs baseline: 2.6891x; 1.1590x over previous
import functools

import jax
import jax.numpy as jnp
from jax import lax
from jax.experimental import pallas as pl
from jax.experimental.pallas import tpu as pltpu
from jax.experimental.pallas import tpu_sc as plsc

_N_CLASS = 100000
_DIM = 16
_N_STEP = 20
_BATCH = 1024

_TOTAL = _BATCH * _N_STEP
_NW = 32
_B_PER_W = _TOTAL // _NW
_CHUNK = 128
_N_CHUNKS = _B_PER_W // _CHUNK


def _make_sc_gather():
    mesh = plsc.VectorSubcoreMesh(core_axis_name="c", subcore_axis_name="s")

    @functools.partial(
        pl.kernel,
        mesh=mesh,
        out_type=jax.ShapeDtypeStruct((_TOTAL, _DIM), jnp.float32),
        scratch_types=[
            pltpu.VMEM((_B_PER_W,), jnp.int32),
            pltpu.VMEM((_B_PER_W, _DIM), jnp.float32),
            pltpu.SemaphoreType.DMA,
        ],
        compiler_params=pltpu.CompilerParams(use_tc_tiling_on_sc=False),
    )
    def gather_kernel(table_hbm, idx_hbm, out_hbm, idx_v, rows_v, sem):
        wid = lax.axis_index("s") * 2 + lax.axis_index("c")
        base = wid * _B_PER_W
        pltpu.sync_copy(idx_hbm.at[pl.ds(base, _B_PER_W)], idx_v)
        copies = []
        for j in range(_N_CHUNKS):
            copies.append(pltpu.async_copy(
                table_hbm.at[idx_v.at[pl.ds(j * _CHUNK, _CHUNK)]],
                rows_v.at[pl.ds(j * _CHUNK, _CHUNK)],
                sem,
            ))
        for c in copies:
            c.wait()
        pltpu.sync_copy(rows_v, out_hbm.at[pl.ds(base, _B_PER_W)])

    return gather_kernel


_sc_gather = _make_sc_gather()


_BN = 1024


def _proj_body(wt_ref, flat_ref, b_ref, out_ref):
    acc = lax.dot_general(
        wt_ref[...], flat_ref[...],
        (((0,), (1,)), ((), ())),
        preferred_element_type=jnp.float32,
    )
    bias_row = b_ref[...].reshape(1, _BN)
    ones_row = jnp.ones((1, _BATCH), jnp.float32)
    out_ref[...] = acc + lax.dot_general(
        bias_row, ones_row,
        (((0,), (0,)), ((), ())),
        preferred_element_type=jnp.float32,
    )


@jax.jit
def _projection(Wt, flat, b):
    nb = pl.cdiv(_N_CLASS, _BN)
    return pl.pallas_call(
        _proj_body,
        grid=(nb,),
        in_specs=[
            pl.BlockSpec((_N_STEP * _DIM, _BN), lambda i: (0, i)),
            pl.BlockSpec((_BATCH, _N_STEP * _DIM), lambda i: (0, 0)),
            pl.BlockSpec((_BN,), lambda i: (i,)),
        ],
        out_specs=pl.BlockSpec((_BN, _BATCH), lambda i: (i, 0)),
        out_shape=jax.ShapeDtypeStruct((_N_CLASS, _BATCH), jnp.float32),
        compiler_params=pltpu.CompilerParams(
            dimension_semantics=("parallel",),
        ),
    )(Wt, flat, b)


def kernel(x, table, W, b):
    idx = x.reshape(-1).astype(jnp.int32)
    rows = _sc_gather(table, idx)
    flat = rows.reshape(_BATCH, _N_STEP * _DIM)
    outT = _projection(W.T, flat, b)
    return outT.T


# speedup vs baseline: 3.0797x; 1.1453x over previous
import functools

import jax
import jax.numpy as jnp
from jax import lax
from jax.experimental import pallas as pl
from jax.experimental.pallas import tpu as pltpu
from jax.experimental.pallas import tpu_sc as plsc

_N_CLASS = 100000
_DIM = 16
_N_STEP = 20
_BATCH = 1024

_TOTAL = _BATCH * _N_STEP
_NW = 32
_B_PER_W = _TOTAL // _NW
_CHUNK = 128
_N_CHUNKS = _B_PER_W // _CHUNK


def _make_sc_gather():
    mesh = plsc.VectorSubcoreMesh(core_axis_name="c", subcore_axis_name="s")

    @functools.partial(
        pl.kernel,
        mesh=mesh,
        out_type=jax.ShapeDtypeStruct((_TOTAL, _DIM), jnp.float32),
        scratch_types=[
            pltpu.VMEM((_B_PER_W,), jnp.int32),
            pltpu.VMEM((_B_PER_W, _DIM), jnp.float32),
            pltpu.SemaphoreType.DMA,
        ],
        compiler_params=pltpu.CompilerParams(use_tc_tiling_on_sc=False),
    )
    def gather_kernel(table_hbm, idx_hbm, out_hbm, idx_v, rows_v, sem):
        wid = lax.axis_index("s") * 2 + lax.axis_index("c")
        base = wid * _B_PER_W
        pltpu.sync_copy(idx_hbm.at[pl.ds(base, _B_PER_W)], idx_v)
        copies = []
        for j in range(_N_CHUNKS):
            copies.append(pltpu.async_copy(
                table_hbm.at[idx_v.at[pl.ds(j * _CHUNK, _CHUNK)]],
                rows_v.at[pl.ds(j * _CHUNK, _CHUNK)],
                sem,
            ))
        for c in copies:
            c.wait()
        pltpu.sync_copy(rows_v, out_hbm.at[pl.ds(base, _B_PER_W)])

    return gather_kernel


_sc_gather = _make_sc_gather()


_BN = 2048


def _proj_body(wt_ref, flat_ref, b_ref, out_ref):
    acc = lax.dot_general(
        wt_ref[...], flat_ref[...],
        (((0,), (1,)), ((), ())),
        preferred_element_type=jnp.float32,
    )
    bias_row = b_ref[...].reshape(1, _BN)
    ones_row = jnp.ones((1, _BATCH), jnp.float32)
    out_ref[...] = acc + lax.dot_general(
        bias_row, ones_row,
        (((0,), (0,)), ((), ())),
        preferred_element_type=jnp.float32,
    )


@jax.jit
def _projection(Wt, flat, b):
    nb = pl.cdiv(_N_CLASS, _BN)
    return pl.pallas_call(
        _proj_body,
        grid=(nb,),
        in_specs=[
            pl.BlockSpec((_N_STEP * _DIM, _BN), lambda i: (0, i)),
            pl.BlockSpec((_BATCH, _N_STEP * _DIM), lambda i: (0, 0)),
            pl.BlockSpec((_BN,), lambda i: (i,)),
        ],
        out_specs=pl.BlockSpec((_BN, _BATCH), lambda i: (i, 0)),
        out_shape=jax.ShapeDtypeStruct((_N_CLASS, _BATCH), jnp.float32),
        compiler_params=pltpu.CompilerParams(
            dimension_semantics=("parallel",),
        ),
    )(Wt, flat, b)


def kernel(x, table, W, b):
    idx = x.reshape(-1).astype(jnp.int32)
    rows = _sc_gather(table, idx)
    flat = rows.reshape(_BATCH, _N_STEP * _DIM)
    outT = _projection(W.T, flat, b)
    return outT.T


# speedup vs baseline: 3.1812x; 1.0329x over previous
import functools

import jax
import jax.numpy as jnp
from jax import lax
from jax.experimental import pallas as pl
from jax.experimental.pallas import tpu as pltpu
from jax.experimental.pallas import tpu_sc as plsc

_N_CLASS = 100000
_DIM = 16
_N_STEP = 20
_BATCH = 1024

_TOTAL = _BATCH * _N_STEP
_NW = 32
_B_PER_W = _TOTAL // _NW
_CHUNK = 128
_N_CHUNKS = _B_PER_W // _CHUNK


def _make_sc_gather():
    mesh = plsc.VectorSubcoreMesh(core_axis_name="c", subcore_axis_name="s")

    @functools.partial(
        pl.kernel,
        mesh=mesh,
        out_type=jax.ShapeDtypeStruct((_TOTAL, _DIM), jnp.float32),
        scratch_types=[
            pltpu.VMEM((_B_PER_W,), jnp.int32),
            pltpu.VMEM((_B_PER_W, _DIM), jnp.float32),
            pltpu.SemaphoreType.DMA,
        ],
        compiler_params=pltpu.CompilerParams(use_tc_tiling_on_sc=False),
    )
    def gather_kernel(table_hbm, idx_hbm, out_hbm, idx_v, rows_v, sem):
        wid = lax.axis_index("s") * 2 + lax.axis_index("c")
        base = wid * _B_PER_W
        pltpu.sync_copy(idx_hbm.at[pl.ds(base, _B_PER_W)], idx_v)
        copies = []
        for j in range(_N_CHUNKS):
            copies.append(pltpu.async_copy(
                table_hbm.at[idx_v.at[pl.ds(j * _CHUNK, _CHUNK)]],
                rows_v.at[pl.ds(j * _CHUNK, _CHUNK)],
                sem,
            ))
        for c in copies:
            c.wait()
        pltpu.sync_copy(rows_v, out_hbm.at[pl.ds(base, _B_PER_W)])

    return gather_kernel


_sc_gather = _make_sc_gather()


_BN = 4096


def _proj_body(wt_ref, flat_ref, b_ref, out_ref):
    acc = lax.dot_general(
        wt_ref[...], flat_ref[...],
        (((0,), (1,)), ((), ())),
        preferred_element_type=jnp.float32,
    )
    bias_row = b_ref[...].reshape(1, _BN)
    ones_row = jnp.ones((1, _BATCH), jnp.float32)
    out_ref[...] = acc + lax.dot_general(
        bias_row, ones_row,
        (((0,), (0,)), ((), ())),
        preferred_element_type=jnp.float32,
    )


@jax.jit
def _projection(Wt, flat, b):
    nb = pl.cdiv(_N_CLASS, _BN)
    return pl.pallas_call(
        _proj_body,
        grid=(nb,),
        in_specs=[
            pl.BlockSpec((_N_STEP * _DIM, _BN), lambda i: (0, i)),
            pl.BlockSpec((_BATCH, _N_STEP * _DIM), lambda i: (0, 0)),
            pl.BlockSpec((_BN,), lambda i: (i,)),
        ],
        out_specs=pl.BlockSpec((_BN, _BATCH), lambda i: (i, 0)),
        out_shape=jax.ShapeDtypeStruct((_N_CLASS, _BATCH), jnp.float32),
        compiler_params=pltpu.CompilerParams(
            dimension_semantics=("parallel",),
        ),
    )(Wt, flat, b)


def kernel(x, table, W, b):
    idx = x.reshape(-1).astype(jnp.int32)
    rows = _sc_gather(table, idx)
    flat = rows.reshape(_BATCH, _N_STEP * _DIM)
    outT = _projection(W.T, flat, b)
    return outT.T


# speedup vs baseline: 3.5191x; 1.1062x over previous
import functools

import jax
import jax.numpy as jnp
from jax import lax
from jax.experimental import pallas as pl
from jax.experimental.pallas import tpu as pltpu
from jax.experimental.pallas import tpu_sc as plsc

_N_CLASS = 100000
_DIM = 16
_N_STEP = 20
_BATCH = 1024

_NW = 32
_ROWS_PER_W = (_N_STEP * _DIM) // _NW
_CHUNK = 128
_N_CHUNKS = _BATCH // _CHUNK


def _make_sc_gather():
    mesh = plsc.VectorSubcoreMesh(core_axis_name="c", subcore_axis_name="s")

    @functools.partial(
        pl.kernel,
        mesh=mesh,
        out_type=jax.ShapeDtypeStruct((_N_STEP * _DIM, _BATCH), jnp.float32),
        scratch_types=[
            pltpu.VMEM((_ROWS_PER_W, _BATCH), jnp.int32),
            pltpu.VMEM((_ROWS_PER_W, _BATCH), jnp.float32),
            pltpu.SemaphoreType.DMA,
            pltpu.SemaphoreType.DMA,
        ],
        compiler_params=pltpu.CompilerParams(use_tc_tiling_on_sc=False),
    )
    def gather_kernel(tflat_hbm, xt_hbm, out_hbm, xt_v, rows_v, sem, sem2):
        wid = lax.axis_index("s") * 2 + lax.axis_index("c")
        base = wid * _ROWS_PER_W
        loads = []
        for k in range(_ROWS_PER_W):
            t = (base + k) // _DIM
            loads.append(pltpu.async_copy(
                xt_hbm.at[pl.ds(t * _BATCH, _BATCH)], xt_v.at[k], sem2))
        for c in loads:
            c.wait()
        copies = []
        for k in range(_ROWS_PER_W):
            d = (base + k) % _DIM
            window = tflat_hbm.at[pl.ds(d * _N_CLASS, _N_CLASS)]
            for j in range(_N_CHUNKS):
                copies.append(pltpu.async_copy(
                    window.at[xt_v.at[k, pl.ds(j * _CHUNK, _CHUNK)]],
                    rows_v.at[k, pl.ds(j * _CHUNK, _CHUNK)],
                    sem,
                ))
        for c in copies:
            c.wait()
        pltpu.sync_copy(rows_v, out_hbm.at[pl.ds(base, _ROWS_PER_W)])

    return gather_kernel


_sc_gather = _make_sc_gather()


_BN = 4096


def _proj_body(wt_ref, flat_ref, b_ref, out_ref):
    acc = lax.dot_general(
        wt_ref[...], flat_ref[...],
        (((0,), (0,)), ((), ())),
        preferred_element_type=jnp.float32,
    )
    bias_row = b_ref[...].reshape(1, _BN)
    ones_row = jnp.ones((1, _BATCH), jnp.float32)
    out_ref[...] = acc + lax.dot_general(
        bias_row, ones_row,
        (((0,), (0,)), ((), ())),
        preferred_element_type=jnp.float32,
    )


@jax.jit
def _projection(Wt, flat, b):
    nb = pl.cdiv(_N_CLASS, _BN)
    return pl.pallas_call(
        _proj_body,
        grid=(nb,),
        in_specs=[
            pl.BlockSpec((_N_STEP * _DIM, _BN), lambda i: (0, i)),
            pl.BlockSpec((_N_STEP * _DIM, _BATCH), lambda i: (0, 0)),
            pl.BlockSpec((_BN,), lambda i: (i,)),
        ],
        out_specs=pl.BlockSpec((_BN, _BATCH), lambda i: (i, 0)),
        out_shape=jax.ShapeDtypeStruct((_N_CLASS, _BATCH), jnp.float32),
        compiler_params=pltpu.CompilerParams(
            dimension_semantics=("parallel",),
        ),
    )(Wt, flat, b)


def kernel(x, table, W, b):
    xt = x.T.reshape(-1).astype(jnp.int32)
    tflat = table.T.reshape(-1)
    flatT = _sc_gather(tflat, xt)
    outT = _projection(W.T, flatT, b)
    return outT.T
